# baseline (speedup 1.0000x reference)
import jax
import jax.numpy as jnp
import numpy as np
from jax import lax
from jax.experimental import pallas as pl
from jax.experimental.pallas import tpu as pltpu
from jax.experimental.pallas import tpu_sc as plsc

N = 10000
E = 320000
F = 128

NC = 2
NS = 16
NW = NC * NS
CK = 128
NCH = 80
IB = 8
EPW = NCH * CK
EP = NW * EPW
NP = 10240
RPS = NP // NS

TCB = 1024

_SC_MESH = plsc.VectorSubcoreMesh(core_axis_name="c", subcore_axis_name="s")


def _agg_body(xl, srcr, dstr, acc_out, src_v, dst_v, rows0, rows1,
              acc_sh, sem0, sem1):
    c = lax.axis_index("c")
    s = lax.axis_index("s")
    wid = c * NS + s
    rows = (rows0, rows1)
    sems = (sem0, sem1)

    @pl.loop(0, CK)
    def _(i):
        @pl.loop(0, F, step=16)
        def _(j):
            rows0[i, pl.ds(j, 16)] = jnp.zeros((16,), jnp.float32)

    @pl.loop(0, RPS, step=CK)
    def _(r):
        pltpu.sync_copy(rows0, acc_sh.at[pl.ds(s * RPS + r, CK)])

    plsc.subcore_barrier()

    @pl.loop(0, NCH, step=IB)
    def _(g):
        pltpu.sync_copy(srcr.at[wid].at[pl.ds(g, IB)], src_v)
        pltpu.sync_copy(dstr.at[wid].at[pl.ds(g, IB)], dst_v)

        desc = pltpu.async_copy(xl.at[src_v.at[0]], rows[0], sems[0])
        for j in range(IB):
            nxt = None
            if j + 1 < IB:
                nxt = pltpu.async_copy(xl.at[src_v.at[j + 1]],
                                       rows[(j + 1) % 2], sems[(j + 1) % 2])
            desc.wait()
            pltpu.sync_copy(rows[j % 2], acc_sh.at[dst_v.at[j]], add=True)
            desc = nxt

    plsc.subcore_barrier()

    pltpu.sync_copy(acc_sh.at[pl.ds(s * RPS, RPS)],
                    acc_out.at[c].at[pl.ds(s * RPS, RPS)])


_agg = pl.kernel(
    _agg_body,
    out_type=jax.ShapeDtypeStruct((NC, NP, F), jnp.float32),
    mesh=_SC_MESH,
    scratch_types=[
        pltpu.VMEM((IB, CK), jnp.int32),
        pltpu.VMEM((IB, CK), jnp.int32),
        pltpu.VMEM((CK, F), jnp.float32),
        pltpu.VMEM((CK, F), jnp.float32),
        pltpu.VMEM_SHARED((NP, F), jnp.float32),
        pltpu.SemaphoreType.DMA,
        pltpu.SemaphoreType.DMA,
    ],
)


def _deg_body(dstr, deg_out, dst_v, ones_v, deg_sh, dsem):
    c = lax.axis_index("c")
    s = lax.axis_index("s")
    wid = c * NS + s

    @pl.loop(0, CK)
    def _(i):
        @pl.loop(0, F, step=16)
        def _(j):
            ones_v[i, pl.ds(j, 16)] = jnp.zeros((16,), jnp.float32)

    @pl.loop(0, RPS, step=CK)
    def _(r):
        pltpu.sync_copy(ones_v, deg_sh.at[pl.ds(s * RPS + r, CK)])

    @pl.loop(0, CK)
    def _(i):
        @pl.loop(0, F, step=16)
        def _(j):
            ones_v[i, pl.ds(j, 16)] = jnp.ones((16,), jnp.float32)

    plsc.subcore_barrier()

    @pl.loop(0, NCH, step=IB)
    def _(g):
        pltpu.sync_copy(dstr.at[wid].at[pl.ds(g, IB)], dst_v)
        descs = [pltpu.async_copy(ones_v, deg_sh.at[dst_v.at[j]], dsem,
                                  add=True)
                 for j in range(IB)]
        for d in descs:
            d.wait()

    plsc.subcore_barrier()

    pltpu.sync_copy(deg_sh.at[pl.ds(s * RPS, RPS)],
                    deg_out.at[c].at[pl.ds(s * RPS, RPS)])


_deg = pl.kernel(
    _deg_body,
    out_type=jax.ShapeDtypeStruct((NC, NP, F), jnp.float32),
    mesh=_SC_MESH,
    scratch_types=[
        pltpu.VMEM((IB, CK), jnp.int32),
        pltpu.VMEM((CK, F), jnp.float32),
        pltpu.VMEM_SHARED((NP, F), jnp.float32),
        pltpu.SemaphoreType.DMA,
    ],
)


def _pre_body(x_ref, wl_ref, wr_ref, b_ref, xl_ref, xr_ref):
    xb = x_ref[...]
    xl_ref[...] = jnp.dot(xb, wl_ref[...], preferred_element_type=jnp.float32,
                          precision=lax.Precision.HIGHEST)
    xr_ref[...] = jnp.dot(xb, wr_ref[...], preferred_element_type=jnp.float32,
                          precision=lax.Precision.HIGHEST) + b_ref[...]


def _mid_body(a_ref, dg_ref, xr1_ref, wl_ref, wr_ref, b_ref,
              xl2_ref, xr2_ref):
    ssum = a_ref[0] + a_ref[1]
    deg = dg_ref[0, :, :1] + dg_ref[1, :, :1]
    h = jnp.maximum(ssum / jnp.maximum(deg, 1.0) + xr1_ref[...], 0.0)
    xl2_ref[...] = jnp.dot(h, wl_ref[...], preferred_element_type=jnp.float32,
                           precision=lax.Precision.HIGHEST)
    xr2_ref[...] = jnp.dot(h, wr_ref[...], preferred_element_type=jnp.float32,
                           precision=lax.Precision.HIGHEST) + b_ref[...]


def _fin_body(a_ref, dg_ref, xr2_ref, out_ref):
    deg = dg_ref[0, :, :1] + dg_ref[1, :, :1]
    t = (a_ref[0] + a_ref[1]) / jnp.maximum(deg, 1.0) + xr2_ref[...]
    nrm = jnp.sqrt(jnp.sum(t * t, axis=1, keepdims=True))
    out_ref[...] = t / jnp.maximum(nrm, 1e-12)


_ROWS = pl.BlockSpec((TCB, F), lambda i: (i, 0))
_PAIR = pl.BlockSpec((NC, TCB, F), lambda i: (0, i, 0))
_WMAT = pl.BlockSpec((F, F), lambda i: (0, 0))
_BVEC = pl.BlockSpec((1, F), lambda i: (0, 0))
_GRID = (NP // TCB,)
_ROWS_OUT = jax.ShapeDtypeStruct((NP, F), jnp.float32)

_pre = pl.pallas_call(
    _pre_body, grid=_GRID,
    in_specs=[_ROWS, _WMAT, _WMAT, _BVEC],
    out_specs=[_ROWS, _ROWS],
    out_shape=[_ROWS_OUT, _ROWS_OUT],
)

_mid = pl.pallas_call(
    _mid_body, grid=_GRID,
    in_specs=[_PAIR, _PAIR, _ROWS, _WMAT, _WMAT, _BVEC],
    out_specs=[_ROWS, _ROWS],
    out_shape=[_ROWS_OUT, _ROWS_OUT],
)

_fin = pl.pallas_call(
    _fin_body, grid=_GRID,
    in_specs=[_PAIR, _PAIR, _ROWS],
    out_specs=_ROWS,
    out_shape=_ROWS_OUT,
)


def kernel(x, edge_index, W1_l, b1, W1_r, W2_l, b2, W2_r):
    xp = jnp.pad(x, ((0, NP - N), (0, 0)))
    src = jnp.concatenate([edge_index[0],
                           jnp.zeros((EP - E,), jnp.int32)]).reshape(NW, NCH, CK)
    dst = jnp.concatenate([edge_index[1],
                           jnp.full((EP - E,), N, jnp.int32)]).reshape(NW, NCH, CK)
    deg = _deg(dst)
    xl1, xr1 = _pre(xp, W1_l.T, W1_r.T, b1.reshape(1, F))
    acc1 = _agg(xl1, src, dst)
    xl2, xr2 = _mid(acc1, deg, xr1, W2_l.T, W2_r.T, b2.reshape(1, F))
    acc2 = _agg(xl2, src, dst)
    return _fin(acc2, deg, xr2)[:N]
